# baseline (speedup 1.0000x reference)
import functools

import jax
import jax.numpy as jnp
from jax import lax
from jax.experimental import pallas as pl
from jax.experimental.pallas import tpu as pltpu
from jax.experimental.pallas import tpu_sc as plsc

NC, NS, LANES = 2, 16, 16
NW = NC * NS
F32 = jnp.float32
I32 = jnp.int32

_SC_PARAMS = pltpu.CompilerParams(needs_layout_passes=False)


def _lrelu(v):
    return jnp.where(v > 0, v, 0.2 * v)


def _mm_body(KT, BN, lhs_ref, rhs_ref, out_ref):
    acc = jnp.zeros((BN, 128), F32)
    for kt in range(KT):
        acc = acc + jnp.dot(lhs_ref[kt], rhs_ref[pl.ds(kt * 128, 128), :],
                            preferred_element_type=F32)
    out_ref[0] = acc


def mm3d(lhs3d, rhs):
    KT, n, _ = lhs3d.shape
    K, F = rhs.shape
    OT = F // 128
    BN = 1000
    return pl.pallas_call(
        functools.partial(_mm_body, KT, BN),
        grid=(OT, n // BN),
        in_specs=[
            pl.BlockSpec((KT, BN, 128), lambda o, b: (0, b, 0)),
            pl.BlockSpec((K, 128), lambda o, b: (0, o)),
        ],
        out_specs=pl.BlockSpec((1, BN, 128), lambda o, b: (o, b, 0)),
        out_shape=jax.ShapeDtypeStruct((OT, n, 128), F32),
    )(lhs3d, rhs)


def _alpha_body(E, s_hbm, src_hbm, dst_hbm, out_hbm, s_v, srcv, dstv,
                w2v, w3v, rv):
    cid = lax.axis_index("c")
    sid = lax.axis_index("s")
    wid = sid * NC + cid
    base = wid * (E // NW)
    pltpu.sync_copy(s_hbm, s_v)

    @pl.loop(0, (E // NW) // 400)
    def _chunk(ci):
        e0 = pl.multiple_of(base + ci * 400, 8)
        pltpu.sync_copy(src_hbm.at[pl.ds(e0, 400)], srcv)
        pltpu.sync_copy(dst_hbm.at[pl.ds(e0, 400)], dstv)

        @pl.loop(0, 25)
        def _grp(g):
            sl = pl.ds(g * 16, 16)
            si = srcv[sl] * 4
            di = dstv[sl] * 4
            u0 = jnp.exp(_lrelu(plsc.load_gather(s_v, [di])))
            u1 = jnp.exp(_lrelu(plsc.load_gather(s_v, [di + 1])))
            u2 = jnp.exp(_lrelu(plsc.load_gather(s_v, [si + 2])))
            u3 = jnp.exp(_lrelu(plsc.load_gather(s_v, [si + 3])))
            inv = 1.0 / (u0 + u1 + u2 + u3)
            w2v[sl] = u2 * inv
            w3v[sl] = u3 * inv
            rv[sl] = inv

        pltpu.sync_copy(w2v, out_hbm.at[pl.ds(e0, 400)])
        pltpu.sync_copy(w3v, out_hbm.at[pl.ds(E + e0, 400)])
        pltpu.sync_copy(rv, out_hbm.at[pl.ds(2 * E + e0, 400)])


def sc_alpha(s4f, src, dst):
    n4 = s4f.shape[0]
    E = src.shape[0]
    mesh = plsc.VectorSubcoreMesh(core_axis_name="c", subcore_axis_name="s")
    f = pl.kernel(
        functools.partial(_alpha_body, E),
        out_type=jax.ShapeDtypeStruct((3 * E,), F32),
        mesh=mesh,
        scratch_types=[
            pltpu.VMEM((n4,), F32),
            pltpu.VMEM((400,), I32),
            pltpu.VMEM((400,), I32),
            pltpu.VMEM((400,), F32),
            pltpu.VMEM((400,), F32),
            pltpu.VMEM((400,), F32),
        ],
        compiler_params=_SC_PARAMS,
    )
    return f(s4f, src, dst)


def _spmm_body(hpt, n, E, h_hbm, wr_hbm, src_hbm, dst_hbm, s_hbm, b_hbm,
               out_hbm, agg_sh, srcb, srcadj, dstb, wb, rows_a, rows_b, fbuf,
               sbuf, bv, sga, sgb, ssa, ssb):
    cid = lax.axis_index("c")
    sid = lax.axis_index("s")
    EW = E // NS
    BE = 4000
    NBLK = EW // BE
    BCH = BE // 80
    NRC = n // 80
    KCH = (NRC + NS - 1) // NS
    rows = (rows_a, rows_b)
    ssems = (ssa, ssb)
    gsems = (sga, sgb)

    def wait_scatter(half):
        pltpu.make_async_copy(rows[half], agg_sh.at[pl.ds(0, 80)],
                              ssems[half]).wait()

    eb = pl.multiple_of(sid * EW, 8)

    @pl.loop(0, 80)
    def _z16(r):
        for v in range(8):
            fbuf[r, pl.ds(v * 16, 16)] = jnp.zeros((16,), F32)

    @pl.loop(0, KCH)
    def _zr(k):
        c = sid + k * NS
        @pl.when(c < NRC)
        def _():
            pltpu.sync_copy(fbuf, agg_sh.at[pl.ds(c * 80, 80)])

    plsc.subcore_barrier()

    @pl.loop(0, NBLK)
    def _rblk(b):
        b0 = pl.multiple_of(2 * E + eb + b * BE, 8)
        pltpu.sync_copy(wr_hbm.at[pl.ds(b0, BE)], wb)
        pltpu.sync_copy(dst_hbm.at[sid * NBLK + b], dstb)

        @pl.loop(0, BCH // 2)
        def _ra(p):
            for half in range(2):
                c = 2 * p + half
                rv = rows[half]
                @pl.when(p > 0)
                def _():
                    wait_scatter(half)

                @pl.loop(0, 5)
                def _fill(g):
                    rvec = wb[pl.ds(c * 80 + g * 16, 16)]
                    for e16 in range(16):
                        rb = rvec[e16] * jnp.ones((16,), F32)
                        for v in range(8):
                            rv[g * 16 + e16, pl.ds(v * 16, 16)] = rb

                pltpu.async_copy(rv, agg_sh.at[dstb.at[c]], ssems[half],
                                 add=True)

        wait_scatter(0)
        wait_scatter(1)

    plsc.subcore_barrier()

    @pl.loop(0, hpt)
    def _etile(t):
        htile = (2 * cid + 1) * hpt + t
        otile = cid * hpt + t
        bb = pl.multiple_of(otile * 128, 8)
        pltpu.sync_copy(b_hbm.at[pl.ds(bb, 128)], bv)

        @pl.loop(0, KCH)
        def _ec(k):
            c = sid + k * NS
            @pl.when(c < NRC)
            def _():
                h0 = pl.multiple_of(htile * n + c * 80, 8)
                pltpu.sync_copy(h_hbm.at[pl.ds(h0, 80)], rows_a)
                s0 = pl.multiple_of(c * 320, 8)
                pltpu.sync_copy(s_hbm.at[pl.ds(s0, 320)], sbuf)
                pltpu.sync_copy(agg_sh.at[pl.ds(c * 80, 80)], fbuf)

                @pl.loop(0, 5)
                def _eg(g):
                    lane = g * 16 + lax.iota(I32, 16)
                    tv = plsc.load_gather(sbuf, [lane * 4 + cid])
                    u = jnp.exp(_lrelu(tv))
                    rr = plsc.load_gather(fbuf, [lane, jnp.zeros((16,), I32)])
                    m = u * rr
                    for e16 in range(16):
                        a = m[e16]
                        e = g * 16 + e16
                        for v in range(8):
                            sl = pl.ds(v * 16, 16)
                            rows_a[e, sl] = jnp.maximum(
                                rows_a[e, sl] * a + bv[sl], 0.0)

                pltpu.sync_copy(rows_a, out_hbm.at[otile, pl.ds(c * 80, 80)])

    plsc.subcore_barrier()

    def wait_gather(half):
        pltpu.make_async_copy(h_hbm.at[pl.ds(0, 80)], rows[half],
                              gsems[half]).wait()

    def scale(half, c):
        rv = rows[half]

        @pl.loop(0, 5, unroll=5)
        def _sc(g):
            avec = wb[pl.ds(c * 80 + g * 16, 16)]
            for e16 in range(16):
                a = avec[e16]
                e = g * 16 + e16
                for v in range(8):
                    sl = pl.ds(v * 16, 16)
                    rv[e, sl] = rv[e, sl] * a

    @pl.loop(0, hpt)
    def _tile(t):
        htile = (2 * cid + 1) * hpt + t
        otile = (2 + cid) * hpt + t
        off = htile * n

        @pl.loop(0, 80)
        def _z(r):
            for v in range(8):
                fbuf[r, pl.ds(v * 16, 16)] = jnp.zeros((16,), F32)

        @pl.loop(0, KCH)
        def _zc(k):
            c = sid + k * NS
            @pl.when(c < NRC)
            def _():
                pltpu.sync_copy(fbuf, agg_sh.at[pl.ds(c * 80, 80)])

        plsc.subcore_barrier()

        @pl.loop(0, NBLK)
        def _blk(b):
            w0 = pl.multiple_of(cid * E + eb + b * BE, 8)
            pltpu.sync_copy(wr_hbm.at[pl.ds(w0, BE)], wb)
            pltpu.sync_copy(dst_hbm.at[sid * NBLK + b], dstb)
            s0 = pl.multiple_of(eb + b * BE, 8)
            pltpu.sync_copy(src_hbm.at[pl.ds(s0, BE)], srcb)

            @pl.loop(0, BE // 16)
            def _adj(g):
                sl = pl.ds(g * 16, 16)
                srcadj[sl] = srcb[sl] + off

            pltpu.async_copy(h_hbm.at[srcadj.at[pl.ds(0, 80)]], rows_a, sga)

            @pl.loop(0, BCH // 2)
            def _p(p):
                c0 = 2 * p
                @pl.when(p > 0)
                def _():
                    wait_scatter(1)
                g1 = pl.multiple_of((c0 + 1) * 80, 8)
                pltpu.async_copy(h_hbm.at[srcadj.at[pl.ds(g1, 80)]],
                                 rows_b, sgb)
                wait_gather(0)
                scale(0, c0)
                pltpu.async_copy(rows_a, agg_sh.at[dstb.at[c0]], ssa,
                                 add=True)
                wait_gather(1)
                scale(1, c0 + 1)
                @pl.when(p < BCH // 2 - 1)
                def _():
                    wait_scatter(0)
                    g2 = pl.multiple_of((c0 + 2) * 80, 8)
                    pltpu.async_copy(h_hbm.at[srcadj.at[pl.ds(g2, 80)]],
                                     rows_a, sga)
                pltpu.async_copy(rows_b, agg_sh.at[dstb.at[c0 + 1]], ssb,
                                 add=True)

            wait_scatter(0)
            wait_scatter(1)

        plsc.subcore_barrier()

        bb = pl.multiple_of(otile * 128, 8)
        pltpu.sync_copy(b_hbm.at[pl.ds(bb, 128)], bv)

        @pl.loop(0, KCH)
        def _f(k):
            c = sid + k * NS
            @pl.when(c < NRC)
            def _():
                pltpu.sync_copy(agg_sh.at[pl.ds(c * 80, 80)], fbuf)

                @pl.loop(0, 80)
                def _fr(r):
                    for v in range(8):
                        sl = pl.ds(v * 16, 16)
                        fbuf[r, sl] = jnp.maximum(fbuf[r, sl] + bv[sl], 0.0)

                pltpu.sync_copy(fbuf, out_hbm.at[otile, pl.ds(c * 80, 80)])

        plsc.subcore_barrier()


def sc_spmm(h3d, wrF, src, dst, s4f, bias, oc):
    FT, n, _ = h3d.shape
    E = src.shape[0]
    hpt = oc // 128
    hf = h3d.reshape(FT * n, 128)
    dst3 = dst.reshape(E // 4000, 50, 80)
    mesh = plsc.VectorSubcoreMesh(core_axis_name="c", subcore_axis_name="s")
    f = pl.kernel(
        functools.partial(_spmm_body, hpt, n, E),
        out_type=jax.ShapeDtypeStruct((FT, n, 128), F32),
        mesh=mesh,
        scratch_types=[
            pltpu.MemorySpace.VMEM_SHARED((n, 128), F32),
            pltpu.VMEM((4000,), I32),
            pltpu.VMEM((4000,), I32),
            pltpu.VMEM((50, 80), I32),
            pltpu.VMEM((4000,), F32),
            pltpu.VMEM((80, 128), F32),
            pltpu.VMEM((80, 128), F32),
            pltpu.VMEM((80, 128), F32),
            pltpu.VMEM((320,), F32),
            pltpu.VMEM((128,), F32),
            pltpu.SemaphoreType.DMA,
            pltpu.SemaphoreType.DMA,
            pltpu.SemaphoreType.DMA,
            pltpu.SemaphoreType.DMA,
        ],
        compiler_params=_SC_PARAMS,
    )
    return f(hf, wrF, src, dst3, s4f, bias)


def _segmax_body(FT, n, h_hbm, off_hbm, g_hbm, offv, idxv, rbuf, accb, sem):
    cid = lax.axis_index("c")
    sid = lax.axis_index("s")
    wid = sid * NC + cid
    pltpu.sync_copy(off_hbm, offv)
    T = (64 * FT) // NW

    @pl.loop(0, T)
    def _t(t):
        tau = t * NW + wid
        seg = lax.rem(tau, 64)
        ct = tau // 64
        ovec = offv[pl.ds(seg, 16)]
        s0 = ovec[0]
        s1 = ovec[1]
        nch = (s1 - s0 + 127) // 128
        for v in range(8):
            accb[pl.ds(v * 16, 16)] = jnp.full((16,), -jnp.inf, F32)

        @pl.loop(0, nch)
        def _c(c):
            rbase = s0 + c * 128

            @pl.loop(0, 8)
            def _ib(g):
                ii = rbase + g * 16 + lax.iota(I32, 16)
                idxv[0, pl.ds(g * 16, 16)] = jnp.minimum(ii, s1 - 1) + ct * n

            pltpu.async_copy(h_hbm.at[idxv.at[0]], rbuf, sem).wait()

            @pl.loop(0, 128)
            def _r(r):
                for v in range(8):
                    sl = pl.ds(v * 16, 16)
                    accb[sl] = jnp.maximum(accb[sl], rbuf[r, sl])

        o0 = pl.multiple_of((ct * 64 + seg) * 128, 8)
        pltpu.sync_copy(accb, g_hbm.at[pl.ds(o0, 128)])


def sc_segmax(x3d, offs):
    FT, n, _ = x3d.shape
    xf = x3d.reshape(FT * n, 128)
    mesh = plsc.VectorSubcoreMesh(core_axis_name="c", subcore_axis_name="s")
    f = pl.kernel(
        functools.partial(_segmax_body, FT, n),
        out_type=jax.ShapeDtypeStruct((FT * 64 * 128,), F32),
        mesh=mesh,
        scratch_types=[
            pltpu.VMEM((80,), I32),
            pltpu.VMEM((1, 128), I32),
            pltpu.VMEM((128, 128), F32),
            pltpu.VMEM((128,), F32),
            pltpu.SemaphoreType.DMA,
        ],
        compiler_params=_SC_PARAMS,
    )
    return f(xf, offs).reshape(FT, 64, 128)


def _mlp_body(FT, g_ref, w1_ref, b1_ref, w2_ref, b2_ref, w3_ref, b3_ref,
              out_ref):
    acc = jnp.zeros((64, w1_ref.shape[1]), F32)
    for ct in range(FT):
        acc = acc + jnp.dot(g_ref[ct], w1_ref[pl.ds(ct * 128, 128), :],
                            preferred_element_type=F32)
    t = jnp.maximum(acc + b1_ref[...], 0.0)
    t2 = jnp.dot(t, w2_ref[...], preferred_element_type=F32) + b2_ref[...]
    y = jnp.dot(t2, w3_ref[...], preferred_element_type=F32) + b3_ref[...]
    out_ref[...] = y


def mlp_head(g3d, fc1_w, fc1_b, fc2_w, fc2_b, out_w_pad, out_b_pad):
    FT = g3d.shape[0]
    return pl.pallas_call(
        functools.partial(_mlp_body, FT),
        out_shape=jax.ShapeDtypeStruct((64, 128), F32),
    )(g3d, fc1_w, fc1_b.reshape(1, -1), fc2_w, fc2_b.reshape(1, -1),
      out_w_pad, out_b_pad)


def _build_A(att, oc):
    a = att[0]
    A = jnp.zeros((4 * oc, 128), F32)
    A = A.at[:2 * oc, 0].set(a[0])
    A = A.at[2 * oc:, 1].set(a[1])
    A = A.at[:2 * oc, 2].set(a[2])
    A = A.at[2 * oc:, 3].set(a[3])
    return A


def _gat_layer(h3d, att, bias, src, dst, oc):
    s4f = mm3d(h3d, _build_A(att, oc))[0, :, :4].reshape(-1)
    wrF = sc_alpha(s4f, src, dst)
    return sc_spmm(h3d, wrF, src, dst, s4f, bias, oc)


def kernel(x, edge_index, batch, W1, a1, b1, W2, a2, b2, W3, a3, b3,
           fc1_w, fc1_b, fc2_w, fc2_b, out_w, out_b):
    n, d = x.shape
    src = edge_index[0]
    dst = edge_index[1]
    x2 = _gat_layer(mm3d(x.reshape(1, n, d), W1), a1, b1, src, dst, d)
    x3 = _gat_layer(mm3d(x2, W2), a2, b2, src, dst, 2 * d)
    x4 = _gat_layer(mm3d(x3, W3), a3, b3, src, dst, 4 * d)

    offs = jnp.searchsorted(batch, jnp.arange(65, dtype=I32)).astype(I32)
    offs = jnp.concatenate([offs, jnp.full((15,), n, I32)])
    g3d = sc_segmax(x4, offs)

    out_w_pad = jnp.zeros((128, 128), F32).at[:, 0].set(out_w[:, 0])
    out_b_pad = jnp.zeros((1, 128), F32).at[0, 0].set(out_b[0])
    y = mlp_head(g3d, fc1_w, fc1_b, fc2_w, fc2_b, out_w_pad, out_b_pad)
    return y[:, :1]


# speedup vs baseline: 1.0627x; 1.0627x over previous
import functools

import jax
import jax.numpy as jnp
from jax import lax
from jax.experimental import pallas as pl
from jax.experimental.pallas import tpu as pltpu
from jax.experimental.pallas import tpu_sc as plsc

NC, NS, LANES = 2, 16, 16
NW = NC * NS
F32 = jnp.float32
I32 = jnp.int32

_SC_PARAMS = pltpu.CompilerParams(needs_layout_passes=False)


def _lrelu(v):
    return jnp.where(v > 0, v, 0.2 * v)


def _mm_body(KT, BN, lhs_ref, rhs_ref, out_ref):
    acc = jnp.zeros((BN, 128), F32)
    for kt in range(KT):
        acc = acc + jnp.dot(lhs_ref[kt], rhs_ref[pl.ds(kt * 128, 128), :],
                            preferred_element_type=F32)
    out_ref[0] = acc


def mm3d(lhs3d, rhs):
    KT, n, _ = lhs3d.shape
    K, F = rhs.shape
    OT = F // 128
    BN = 1000
    return pl.pallas_call(
        functools.partial(_mm_body, KT, BN),
        grid=(OT, n // BN),
        in_specs=[
            pl.BlockSpec((KT, BN, 128), lambda o, b: (0, b, 0)),
            pl.BlockSpec((K, 128), lambda o, b: (0, o)),
        ],
        out_specs=pl.BlockSpec((1, BN, 128), lambda o, b: (o, b, 0)),
        out_shape=jax.ShapeDtypeStruct((OT, n, 128), F32),
    )(lhs3d, rhs)


def _alpha_body(E, s_hbm, src_hbm, dst_hbm, out_hbm, s_v, srcv, dstv,
                w2v, w3v, rv):
    cid = lax.axis_index("c")
    sid = lax.axis_index("s")
    wid = sid * NC + cid
    base = wid * (E // NW)
    pltpu.sync_copy(s_hbm, s_v)

    @pl.loop(0, (E // NW) // 400)
    def _chunk(ci):
        e0 = pl.multiple_of(base + ci * 400, 8)
        pltpu.sync_copy(src_hbm.at[pl.ds(e0, 400)], srcv)
        pltpu.sync_copy(dst_hbm.at[pl.ds(e0, 400)], dstv)

        @pl.loop(0, 25)
        def _grp(g):
            sl = pl.ds(g * 16, 16)
            si = srcv[sl] * 4
            di = dstv[sl] * 4
            u0 = jnp.exp(_lrelu(plsc.load_gather(s_v, [di])))
            u1 = jnp.exp(_lrelu(plsc.load_gather(s_v, [di + 1])))
            u2 = jnp.exp(_lrelu(plsc.load_gather(s_v, [si + 2])))
            u3 = jnp.exp(_lrelu(plsc.load_gather(s_v, [si + 3])))
            inv = 1.0 / (u0 + u1 + u2 + u3)
            w2v[sl] = u2 * inv
            w3v[sl] = u3 * inv
            rv[sl] = inv

        pltpu.sync_copy(w2v, out_hbm.at[pl.ds(e0, 400)])
        pltpu.sync_copy(w3v, out_hbm.at[pl.ds(E + e0, 400)])
        pltpu.sync_copy(rv, out_hbm.at[pl.ds(2 * E + e0, 400)])


def sc_alpha(s4f, src, dst):
    n4 = s4f.shape[0]
    E = src.shape[0]
    mesh = plsc.VectorSubcoreMesh(core_axis_name="c", subcore_axis_name="s")
    f = pl.kernel(
        functools.partial(_alpha_body, E),
        out_type=jax.ShapeDtypeStruct((3 * E,), F32),
        mesh=mesh,
        scratch_types=[
            pltpu.VMEM((n4,), F32),
            pltpu.VMEM((400,), I32),
            pltpu.VMEM((400,), I32),
            pltpu.VMEM((400,), F32),
            pltpu.VMEM((400,), F32),
            pltpu.VMEM((400,), F32),
        ],
        compiler_params=_SC_PARAMS,
    )
    return f(s4f, src, dst)


def _spmm_body(hpt, n, E, h_hbm, wr_hbm, src_hbm, dst_hbm, s_hbm, b_hbm,
               out_hbm, agg_sh, srcb, srcadj, dstb, wb, rows_a, rows_b, fbuf,
               sbuf, bv, sga, sgb, ssa, ssb):
    cid = lax.axis_index("c")
    sid = lax.axis_index("s")
    EW = E // NS
    BE = 4000
    NBLK = EW // BE
    BCH = BE // 80
    NRC = n // 80
    KCH = (NRC + NS - 1) // NS
    rows = (rows_a, rows_b)
    ssems = (ssa, ssb)
    gsems = (sga, sgb)

    def wait_scatter(half):
        pltpu.make_async_copy(rows[half], agg_sh.at[pl.ds(0, 80)],
                              ssems[half]).wait()

    eb = pl.multiple_of(sid * EW, 8)

    @pl.loop(0, 80)
    def _z16(r):
        for v in range(8):
            fbuf[r, pl.ds(v * 16, 16)] = jnp.zeros((16,), F32)

    @pl.loop(0, KCH)
    def _zr(k):
        c = sid + k * NS
        @pl.when(c < NRC)
        def _():
            pltpu.sync_copy(fbuf, agg_sh.at[pl.ds(c * 80, 80)])

    plsc.subcore_barrier()

    @pl.loop(0, NBLK)
    def _rblk(b):
        b0 = pl.multiple_of(2 * E + eb + b * BE, 8)
        pltpu.sync_copy(wr_hbm.at[pl.ds(b0, BE)], wb)
        pltpu.sync_copy(dst_hbm.at[sid * NBLK + b], dstb)

        @pl.loop(0, BCH // 2)
        def _ra(p):
            for half in range(2):
                c = 2 * p + half
                rv = rows[half]
                @pl.when(p > 0)
                def _():
                    wait_scatter(half)

                @pl.loop(0, 5)
                def _fill(g):
                    rvec = wb[pl.ds(c * 80 + g * 16, 16)]
                    for e16 in range(16):
                        rb = rvec[e16] * jnp.ones((16,), F32)
                        for v in range(8):
                            rv[g * 16 + e16, pl.ds(v * 16, 16)] = rb

                pltpu.async_copy(rv, agg_sh.at[dstb.at[c]], ssems[half],
                                 add=True)

        wait_scatter(0)
        wait_scatter(1)

    plsc.subcore_barrier()

    @pl.loop(0, hpt)
    def _etile(t):
        htile = (2 * cid + 1) * hpt + t
        otile = cid * hpt + t
        bb = pl.multiple_of(otile * 128, 8)
        pltpu.sync_copy(b_hbm.at[pl.ds(bb, 128)], bv)

        @pl.loop(0, KCH)
        def _ec(k):
            c = sid + k * NS
            @pl.when(c < NRC)
            def _():
                h0 = pl.multiple_of(htile * n + c * 80, 8)
                pltpu.sync_copy(h_hbm.at[pl.ds(h0, 80)], rows_a)
                s0 = pl.multiple_of(c * 320, 8)
                pltpu.sync_copy(s_hbm.at[pl.ds(s0, 320)], sbuf)
                pltpu.sync_copy(agg_sh.at[pl.ds(c * 80, 80)], fbuf)

                @pl.loop(0, 5)
                def _eg(g):
                    lane = g * 16 + lax.iota(I32, 16)
                    tv = plsc.load_gather(sbuf, [lane * 4 + cid])
                    u = jnp.exp(_lrelu(tv))
                    rr = plsc.load_gather(fbuf, [lane, jnp.zeros((16,), I32)])
                    m = u * rr
                    for e16 in range(16):
                        a = m[e16]
                        e = g * 16 + e16
                        for v in range(8):
                            sl = pl.ds(v * 16, 16)
                            rows_a[e, sl] = jnp.maximum(
                                rows_a[e, sl] * a + bv[sl], 0.0)

                pltpu.sync_copy(rows_a, out_hbm.at[otile, pl.ds(c * 80, 80)])

    plsc.subcore_barrier()

    def wait_gather(half):
        pltpu.make_async_copy(h_hbm.at[pl.ds(0, 80)], rows[half],
                              gsems[half]).wait()

    def scale(half, c):
        rv = rows[half]

        @pl.loop(0, 5)
        def _sc(g):
            avec = wb[pl.ds(c * 80 + g * 16, 16)]
            for e16 in range(16):
                a = avec[e16]
                e = g * 16 + e16
                for v in range(8):
                    sl = pl.ds(v * 16, 16)
                    rv[e, sl] = rv[e, sl] * a

    @pl.loop(0, hpt)
    def _tile(t):
        htile = (2 * cid + 1) * hpt + t
        otile = (2 + cid) * hpt + t
        off = htile * n

        @pl.loop(0, 80)
        def _z(r):
            for v in range(8):
                fbuf[r, pl.ds(v * 16, 16)] = jnp.zeros((16,), F32)

        @pl.loop(0, KCH)
        def _zc(k):
            c = sid + k * NS
            @pl.when(c < NRC)
            def _():
                pltpu.sync_copy(fbuf, agg_sh.at[pl.ds(c * 80, 80)])

        plsc.subcore_barrier()

        @pl.loop(0, NBLK)
        def _blk(b):
            w0 = pl.multiple_of(cid * E + eb + b * BE, 8)
            pltpu.sync_copy(wr_hbm.at[pl.ds(w0, BE)], wb)
            pltpu.sync_copy(dst_hbm.at[sid * NBLK + b], dstb)
            s0 = pl.multiple_of(eb + b * BE, 8)
            pltpu.sync_copy(src_hbm.at[pl.ds(s0, BE)], srcb)

            @pl.loop(0, BE // 16)
            def _adj(g):
                sl = pl.ds(g * 16, 16)
                srcadj[sl] = srcb[sl] + off

            pltpu.async_copy(h_hbm.at[srcadj.at[pl.ds(0, 80)]], rows_a, sga)

            @pl.loop(0, BCH // 2)
            def _p(p):
                c0 = 2 * p
                @pl.when(p > 0)
                def _():
                    wait_scatter(1)
                g1 = pl.multiple_of((c0 + 1) * 80, 8)
                pltpu.async_copy(h_hbm.at[srcadj.at[pl.ds(g1, 80)]],
                                 rows_b, sgb)
                wait_gather(0)
                scale(0, c0)
                pltpu.async_copy(rows_a, agg_sh.at[dstb.at[c0]], ssa,
                                 add=True)
                wait_gather(1)
                scale(1, c0 + 1)
                @pl.when(p < BCH // 2 - 1)
                def _():
                    wait_scatter(0)
                    g2 = pl.multiple_of((c0 + 2) * 80, 8)
                    pltpu.async_copy(h_hbm.at[srcadj.at[pl.ds(g2, 80)]],
                                     rows_a, sga)
                pltpu.async_copy(rows_b, agg_sh.at[dstb.at[c0 + 1]], ssb,
                                 add=True)

            wait_scatter(0)
            wait_scatter(1)

        plsc.subcore_barrier()

        bb = pl.multiple_of(otile * 128, 8)
        pltpu.sync_copy(b_hbm.at[pl.ds(bb, 128)], bv)

        @pl.loop(0, KCH)
        def _f(k):
            c = sid + k * NS
            @pl.when(c < NRC)
            def _():
                pltpu.sync_copy(agg_sh.at[pl.ds(c * 80, 80)], fbuf)

                @pl.loop(0, 80)
                def _fr(r):
                    for v in range(8):
                        sl = pl.ds(v * 16, 16)
                        fbuf[r, sl] = jnp.maximum(fbuf[r, sl] + bv[sl], 0.0)

                pltpu.sync_copy(fbuf, out_hbm.at[otile, pl.ds(c * 80, 80)])

        plsc.subcore_barrier()


def sc_spmm(h3d, wrF, src, dst, s4f, bias, oc):
    FT, n, _ = h3d.shape
    E = src.shape[0]
    hpt = oc // 128
    hf = h3d.reshape(FT * n, 128)
    dst3 = dst.reshape(E // 4000, 50, 80)
    mesh = plsc.VectorSubcoreMesh(core_axis_name="c", subcore_axis_name="s")
    f = pl.kernel(
        functools.partial(_spmm_body, hpt, n, E),
        out_type=jax.ShapeDtypeStruct((FT, n, 128), F32),
        mesh=mesh,
        scratch_types=[
            pltpu.MemorySpace.VMEM_SHARED((n, 128), F32),
            pltpu.VMEM((4000,), I32),
            pltpu.VMEM((4000,), I32),
            pltpu.VMEM((50, 80), I32),
            pltpu.VMEM((4000,), F32),
            pltpu.VMEM((80, 128), F32),
            pltpu.VMEM((80, 128), F32),
            pltpu.VMEM((80, 128), F32),
            pltpu.VMEM((320,), F32),
            pltpu.VMEM((128,), F32),
            pltpu.SemaphoreType.DMA,
            pltpu.SemaphoreType.DMA,
            pltpu.SemaphoreType.DMA,
            pltpu.SemaphoreType.DMA,
        ],
        compiler_params=_SC_PARAMS,
    )
    return f(hf, wrF, src, dst3, s4f, bias)


def _segmax_body(FT, n, h_hbm, off_hbm, g_hbm, offv, idxv, rbuf, accb, sem):
    cid = lax.axis_index("c")
    sid = lax.axis_index("s")
    wid = sid * NC + cid
    pltpu.sync_copy(off_hbm, offv)
    T = (64 * FT) // NW

    @pl.loop(0, T)
    def _t(t):
        tau = t * NW + wid
        seg = lax.rem(tau, 64)
        ct = tau // 64
        ovec = offv[pl.ds(seg, 16)]
        s0 = ovec[0]
        s1 = ovec[1]
        nch = (s1 - s0 + 127) // 128
        for v in range(8):
            accb[pl.ds(v * 16, 16)] = jnp.full((16,), -jnp.inf, F32)

        @pl.loop(0, nch)
        def _c(c):
            rbase = s0 + c * 128

            @pl.loop(0, 8)
            def _ib(g):
                ii = rbase + g * 16 + lax.iota(I32, 16)
                idxv[0, pl.ds(g * 16, 16)] = jnp.minimum(ii, s1 - 1) + ct * n

            pltpu.async_copy(h_hbm.at[idxv.at[0]], rbuf, sem).wait()

            @pl.loop(0, 128)
            def _r(r):
                for v in range(8):
                    sl = pl.ds(v * 16, 16)
                    accb[sl] = jnp.maximum(accb[sl], rbuf[r, sl])

        o0 = pl.multiple_of((ct * 64 + seg) * 128, 8)
        pltpu.sync_copy(accb, g_hbm.at[pl.ds(o0, 128)])


def sc_segmax(x3d, offs):
    FT, n, _ = x3d.shape
    xf = x3d.reshape(FT * n, 128)
    mesh = plsc.VectorSubcoreMesh(core_axis_name="c", subcore_axis_name="s")
    f = pl.kernel(
        functools.partial(_segmax_body, FT, n),
        out_type=jax.ShapeDtypeStruct((FT * 64 * 128,), F32),
        mesh=mesh,
        scratch_types=[
            pltpu.VMEM((80,), I32),
            pltpu.VMEM((1, 128), I32),
            pltpu.VMEM((128, 128), F32),
            pltpu.VMEM((128,), F32),
            pltpu.SemaphoreType.DMA,
        ],
        compiler_params=_SC_PARAMS,
    )
    return f(xf, offs).reshape(FT, 64, 128)


def _mlp_body(FT, g_ref, w1_ref, b1_ref, w2_ref, b2_ref, w3_ref, b3_ref,
              out_ref):
    acc = jnp.zeros((64, w1_ref.shape[1]), F32)
    for ct in range(FT):
        acc = acc + jnp.dot(g_ref[ct], w1_ref[pl.ds(ct * 128, 128), :],
                            preferred_element_type=F32)
    t = jnp.maximum(acc + b1_ref[...], 0.0)
    t2 = jnp.dot(t, w2_ref[...], preferred_element_type=F32) + b2_ref[...]
    y = jnp.dot(t2, w3_ref[...], preferred_element_type=F32) + b3_ref[...]
    out_ref[...] = y


def mlp_head(g3d, fc1_w, fc1_b, fc2_w, fc2_b, out_w_pad, out_b_pad):
    FT = g3d.shape[0]
    return pl.pallas_call(
        functools.partial(_mlp_body, FT),
        out_shape=jax.ShapeDtypeStruct((64, 128), F32),
    )(g3d, fc1_w, fc1_b.reshape(1, -1), fc2_w, fc2_b.reshape(1, -1),
      out_w_pad, out_b_pad)


def _build_A(att, oc):
    a = att[0]
    A = jnp.zeros((4 * oc, 128), F32)
    A = A.at[:2 * oc, 0].set(a[0])
    A = A.at[2 * oc:, 1].set(a[1])
    A = A.at[:2 * oc, 2].set(a[2])
    A = A.at[2 * oc:, 3].set(a[3])
    return A


def _gat_layer(h3d, att, bias, src, dst, oc):
    s4f = mm3d(h3d, _build_A(att, oc))[0, :, :4].reshape(-1)
    wrF = sc_alpha(s4f, src, dst)
    return sc_spmm(h3d, wrF, src, dst, s4f, bias, oc)


def kernel(x, edge_index, batch, W1, a1, b1, W2, a2, b2, W3, a3, b3,
           fc1_w, fc1_b, fc2_w, fc2_b, out_w, out_b):
    n, d = x.shape
    src = edge_index[0]
    dst = edge_index[1]
    x2 = _gat_layer(mm3d(x.reshape(1, n, d), W1), a1, b1, src, dst, d)
    x3 = _gat_layer(mm3d(x2, W2), a2, b2, src, dst, 2 * d)
    x4 = _gat_layer(mm3d(x3, W3), a3, b3, src, dst, 4 * d)

    offs = jnp.searchsorted(batch, jnp.arange(65, dtype=I32)).astype(I32)
    offs = jnp.concatenate([offs, jnp.full((15,), n, I32)])
    g3d = sc_segmax(x4, offs)

    out_w_pad = jnp.zeros((128, 128), F32).at[:, 0].set(out_w[:, 0])
    out_b_pad = jnp.zeros((1, 128), F32).at[0, 0].set(out_b[0])
    y = mlp_head(g3d, fc1_w, fc1_b, fc2_w, fc2_b, out_w_pad, out_b_pad)
    return y[:, :1]


# speedup vs baseline: 1.1244x; 1.0581x over previous
import functools

import jax
import jax.numpy as jnp
from jax import lax
from jax.experimental import pallas as pl
from jax.experimental.pallas import tpu as pltpu
from jax.experimental.pallas import tpu_sc as plsc

NC, NS, LANES = 2, 16, 16
NW = NC * NS
F32 = jnp.float32
I32 = jnp.int32

_SC_PARAMS = pltpu.CompilerParams(needs_layout_passes=False)


def _lrelu(v):
    return jnp.where(v > 0, v, 0.2 * v)


def _mm_body(KT, BN, lhs_ref, rhs_ref, out_ref):
    acc = jnp.zeros((BN, 128), F32)
    for kt in range(KT):
        acc = acc + jnp.dot(lhs_ref[kt], rhs_ref[pl.ds(kt * 128, 128), :],
                            preferred_element_type=F32)
    out_ref[0] = acc


def mm3d(lhs3d, rhs):
    KT, n, _ = lhs3d.shape
    K, F = rhs.shape
    OT = F // 128
    BN = 1000
    return pl.pallas_call(
        functools.partial(_mm_body, KT, BN),
        grid=(OT, n // BN),
        in_specs=[
            pl.BlockSpec((KT, BN, 128), lambda o, b: (0, b, 0)),
            pl.BlockSpec((K, 128), lambda o, b: (0, o)),
        ],
        out_specs=pl.BlockSpec((1, BN, 128), lambda o, b: (o, b, 0)),
        out_shape=jax.ShapeDtypeStruct((OT, n, 128), F32),
    )(lhs3d, rhs)


def _alpha_body(E, s_hbm, src_hbm, dst_hbm, out_hbm, s_v, srcv, dstv,
                w2v, w3v, rv):
    cid = lax.axis_index("c")
    sid = lax.axis_index("s")
    wid = sid * NC + cid
    base = wid * (E // NW)
    pltpu.sync_copy(s_hbm, s_v)

    @pl.loop(0, (E // NW) // 400)
    def _chunk(ci):
        e0 = pl.multiple_of(base + ci * 400, 8)
        pltpu.sync_copy(src_hbm.at[pl.ds(e0, 400)], srcv)
        pltpu.sync_copy(dst_hbm.at[pl.ds(e0, 400)], dstv)

        @pl.loop(0, 25)
        def _grp(g):
            sl = pl.ds(g * 16, 16)
            si = srcv[sl] * 4
            di = dstv[sl] * 4
            u0 = jnp.exp(_lrelu(plsc.load_gather(s_v, [di])))
            u1 = jnp.exp(_lrelu(plsc.load_gather(s_v, [di + 1])))
            u2 = jnp.exp(_lrelu(plsc.load_gather(s_v, [si + 2])))
            u3 = jnp.exp(_lrelu(plsc.load_gather(s_v, [si + 3])))
            inv = 1.0 / (u0 + u1 + u2 + u3)
            w2v[sl] = u2 * inv
            w3v[sl] = u3 * inv
            rv[sl] = inv

        pltpu.sync_copy(w2v, out_hbm.at[pl.ds(e0, 400)])
        pltpu.sync_copy(w3v, out_hbm.at[pl.ds(E + e0, 400)])
        pltpu.sync_copy(rv, out_hbm.at[pl.ds(2 * E + e0, 400)])


def sc_alpha(s4f, src, dst):
    n4 = s4f.shape[0]
    E = src.shape[0]
    mesh = plsc.VectorSubcoreMesh(core_axis_name="c", subcore_axis_name="s")
    f = pl.kernel(
        functools.partial(_alpha_body, E),
        out_type=jax.ShapeDtypeStruct((3 * E,), F32),
        mesh=mesh,
        scratch_types=[
            pltpu.VMEM((n4,), F32),
            pltpu.VMEM((400,), I32),
            pltpu.VMEM((400,), I32),
            pltpu.VMEM((400,), F32),
            pltpu.VMEM((400,), F32),
            pltpu.VMEM((400,), F32),
        ],
        compiler_params=_SC_PARAMS,
    )
    return f(s4f, src, dst)


def _spmm_body(hpt, n, E, h_hbm, wr_hbm, src_hbm, dst_hbm, s_hbm, b_hbm,
               out_hbm, agg_sh, srcb, srcadj, dstb, wb, rows_a, rows_b, fbuf,
               sbuf, bv, sga, sgb, ssa, ssb):
    cid = lax.axis_index("c")
    sid = lax.axis_index("s")
    EW = E // NS
    BE = 4000
    NBLK = EW // BE
    BCH = BE // 80
    NRC = n // 80
    KCH = (NRC + NS - 1) // NS
    rows = (rows_a, rows_b)
    ssems = (ssa, ssb)
    gsems = (sga, sgb)

    def wait_scatter(half):
        pltpu.make_async_copy(rows[half], agg_sh.at[pl.ds(0, 80)],
                              ssems[half]).wait()

    eb = pl.multiple_of(sid * EW, 8)

    @pl.loop(0, 80)
    def _z16(r):
        for v in range(8):
            fbuf[r, pl.ds(v * 16, 16)] = jnp.zeros((16,), F32)

    @pl.loop(0, KCH)
    def _zr(k):
        c = sid + k * NS
        @pl.when(c < NRC)
        def _():
            pltpu.sync_copy(fbuf, agg_sh.at[pl.ds(c * 80, 80)])

    plsc.subcore_barrier()

    @pl.loop(0, NBLK)
    def _rblk(b):
        b0 = pl.multiple_of(2 * E + eb + b * BE, 8)
        pltpu.sync_copy(wr_hbm.at[pl.ds(b0, BE)], wb)
        pltpu.sync_copy(dst_hbm.at[sid * NBLK + b], dstb)

        @pl.loop(0, BCH // 2)
        def _ra(p):
            for half in range(2):
                c = 2 * p + half
                rv = rows[half]
                @pl.when(p > 0)
                def _():
                    wait_scatter(half)

                @pl.loop(0, 5)
                def _fill(g):
                    rvec = wb[pl.ds(c * 80 + g * 16, 16)]
                    for e16 in range(16):
                        rb = rvec[e16] * jnp.ones((16,), F32)
                        for v in range(8):
                            rv[g * 16 + e16, pl.ds(v * 16, 16)] = rb

                pltpu.async_copy(rv, agg_sh.at[dstb.at[c]], ssems[half],
                                 add=True)

        wait_scatter(0)
        wait_scatter(1)

    plsc.subcore_barrier()

    @pl.loop(0, hpt)
    def _etile(t):
        htile = (2 * cid + 1) * hpt + t
        otile = cid * hpt + t
        bb = pl.multiple_of(otile * 128, 8)
        pltpu.sync_copy(b_hbm.at[pl.ds(bb, 128)], bv)

        @pl.loop(0, KCH)
        def _ec(k):
            c = sid + k * NS
            @pl.when(c < NRC)
            def _():
                h0 = pl.multiple_of(htile * n + c * 80, 8)
                pltpu.sync_copy(h_hbm.at[pl.ds(h0, 80)], rows_a)
                s0 = pl.multiple_of(c * 320, 8)
                pltpu.sync_copy(s_hbm.at[pl.ds(s0, 320)], sbuf)
                pltpu.sync_copy(agg_sh.at[pl.ds(c * 80, 80)], fbuf)

                @pl.loop(0, 5)
                def _eg(g):
                    lane = g * 16 + lax.iota(I32, 16)
                    tv = plsc.load_gather(sbuf, [lane * 4 + cid])
                    u = jnp.exp(_lrelu(tv))
                    rr = plsc.load_gather(fbuf, [lane, jnp.zeros((16,), I32)])
                    m = u * rr
                    for e16 in range(16):
                        a = m[e16]
                        e = g * 16 + e16
                        for v in range(8):
                            sl = pl.ds(v * 16, 16)
                            rows_a[e, sl] = jnp.maximum(
                                rows_a[e, sl] * a + bv[sl], 0.0)

                pltpu.sync_copy(rows_a, out_hbm.at[otile, pl.ds(c * 80, 80)])

    plsc.subcore_barrier()

    def wait_gather(half):
        pltpu.make_async_copy(h_hbm.at[pl.ds(0, 80)], rows[half],
                              gsems[half]).wait()

    def scale(half, c):
        rv = rows[half]

        @plsc.parallel_loop(0, 5)
        def _sc(g):
            avec = wb[pl.ds(c * 80 + g * 16, 16)]
            for e16 in range(16):
                a = avec[e16]
                e = g * 16 + e16
                for v in range(8):
                    sl = pl.ds(v * 16, 16)
                    rv[e, sl] = rv[e, sl] * a

    @pl.loop(0, hpt)
    def _tile(t):
        htile = (2 * cid + 1) * hpt + t
        otile = (2 + cid) * hpt + t
        off = htile * n

        @pl.loop(0, 80)
        def _z(r):
            for v in range(8):
                fbuf[r, pl.ds(v * 16, 16)] = jnp.zeros((16,), F32)

        @pl.loop(0, KCH)
        def _zc(k):
            c = sid + k * NS
            @pl.when(c < NRC)
            def _():
                pltpu.sync_copy(fbuf, agg_sh.at[pl.ds(c * 80, 80)])

        plsc.subcore_barrier()

        @pl.loop(0, NBLK)
        def _blk(b):
            w0 = pl.multiple_of(cid * E + eb + b * BE, 8)
            pltpu.sync_copy(wr_hbm.at[pl.ds(w0, BE)], wb)
            pltpu.sync_copy(dst_hbm.at[sid * NBLK + b], dstb)
            s0 = pl.multiple_of(eb + b * BE, 8)
            pltpu.sync_copy(src_hbm.at[pl.ds(s0, BE)], srcb)

            @pl.loop(0, BE // 16)
            def _adj(g):
                sl = pl.ds(g * 16, 16)
                srcadj[sl] = srcb[sl] + off

            pltpu.async_copy(h_hbm.at[srcadj.at[pl.ds(0, 80)]], rows_a, sga)

            @pl.loop(0, BCH // 2)
            def _p(p):
                c0 = 2 * p
                @pl.when(p > 0)
                def _():
                    wait_scatter(1)
                g1 = pl.multiple_of((c0 + 1) * 80, 8)
                pltpu.async_copy(h_hbm.at[srcadj.at[pl.ds(g1, 80)]],
                                 rows_b, sgb)
                wait_gather(0)
                scale(0, c0)
                pltpu.async_copy(rows_a, agg_sh.at[dstb.at[c0]], ssa,
                                 add=True)
                wait_gather(1)
                scale(1, c0 + 1)
                @pl.when(p < BCH // 2 - 1)
                def _():
                    wait_scatter(0)
                    g2 = pl.multiple_of((c0 + 2) * 80, 8)
                    pltpu.async_copy(h_hbm.at[srcadj.at[pl.ds(g2, 80)]],
                                     rows_a, sga)
                pltpu.async_copy(rows_b, agg_sh.at[dstb.at[c0 + 1]], ssb,
                                 add=True)

            wait_scatter(0)
            wait_scatter(1)

        plsc.subcore_barrier()

        bb = pl.multiple_of(otile * 128, 8)
        pltpu.sync_copy(b_hbm.at[pl.ds(bb, 128)], bv)

        @pl.loop(0, KCH)
        def _f(k):
            c = sid + k * NS
            @pl.when(c < NRC)
            def _():
                pltpu.sync_copy(agg_sh.at[pl.ds(c * 80, 80)], fbuf)

                @pl.loop(0, 80)
                def _fr(r):
                    for v in range(8):
                        sl = pl.ds(v * 16, 16)
                        fbuf[r, sl] = jnp.maximum(fbuf[r, sl] + bv[sl], 0.0)

                pltpu.sync_copy(fbuf, out_hbm.at[otile, pl.ds(c * 80, 80)])

        plsc.subcore_barrier()


def sc_spmm(h3d, wrF, src, dst, s4f, bias, oc):
    FT, n, _ = h3d.shape
    E = src.shape[0]
    hpt = oc // 128
    hf = h3d.reshape(FT * n, 128)
    dst3 = dst.reshape(E // 4000, 50, 80)
    mesh = plsc.VectorSubcoreMesh(core_axis_name="c", subcore_axis_name="s")
    f = pl.kernel(
        functools.partial(_spmm_body, hpt, n, E),
        out_type=jax.ShapeDtypeStruct((FT, n, 128), F32),
        mesh=mesh,
        scratch_types=[
            pltpu.MemorySpace.VMEM_SHARED((n, 128), F32),
            pltpu.VMEM((4000,), I32),
            pltpu.VMEM((4000,), I32),
            pltpu.VMEM((50, 80), I32),
            pltpu.VMEM((4000,), F32),
            pltpu.VMEM((80, 128), F32),
            pltpu.VMEM((80, 128), F32),
            pltpu.VMEM((80, 128), F32),
            pltpu.VMEM((320,), F32),
            pltpu.VMEM((128,), F32),
            pltpu.SemaphoreType.DMA,
            pltpu.SemaphoreType.DMA,
            pltpu.SemaphoreType.DMA,
            pltpu.SemaphoreType.DMA,
        ],
        compiler_params=_SC_PARAMS,
    )
    return f(hf, wrF, src, dst3, s4f, bias)


def _segmax_body(FT, n, h_hbm, off_hbm, g_hbm, offv, idxv, rbuf, accb, sem):
    cid = lax.axis_index("c")
    sid = lax.axis_index("s")
    wid = sid * NC + cid
    pltpu.sync_copy(off_hbm, offv)
    T = (64 * FT) // NW

    @pl.loop(0, T)
    def _t(t):
        tau = t * NW + wid
        seg = lax.rem(tau, 64)
        ct = tau // 64
        ovec = offv[pl.ds(seg, 16)]
        s0 = ovec[0]
        s1 = ovec[1]
        nch = (s1 - s0 + 63) // 64
        for v in range(8):
            accb[pl.ds(v * 16, 16)] = jnp.full((16,), -jnp.inf, F32)

        @pl.loop(0, nch)
        def _c(c):
            rbase = s0 + c * 64

            @pl.loop(0, 4)
            def _ib(g):
                ii = rbase + g * 16 + lax.iota(I32, 16)
                idxv[0, pl.ds(g * 16, 16)] = jnp.minimum(ii, s1 - 1) + ct * n

            pltpu.async_copy(h_hbm.at[idxv.at[0]], rbuf, sem).wait()

            @pl.loop(0, 64)
            def _r(r):
                for v in range(8):
                    sl = pl.ds(v * 16, 16)
                    accb[sl] = jnp.maximum(accb[sl], rbuf[r, sl])

        o0 = pl.multiple_of((ct * 64 + seg) * 128, 8)
        pltpu.sync_copy(accb, g_hbm.at[pl.ds(o0, 128)])


def sc_segmax(x3d, offs):
    FT, n, _ = x3d.shape
    xf = x3d.reshape(FT * n, 128)
    mesh = plsc.VectorSubcoreMesh(core_axis_name="c", subcore_axis_name="s")
    f = pl.kernel(
        functools.partial(_segmax_body, FT, n),
        out_type=jax.ShapeDtypeStruct((FT * 64 * 128,), F32),
        mesh=mesh,
        scratch_types=[
            pltpu.VMEM((80,), I32),
            pltpu.VMEM((1, 64), I32),
            pltpu.VMEM((64, 128), F32),
            pltpu.VMEM((128,), F32),
            pltpu.SemaphoreType.DMA,
        ],
        compiler_params=_SC_PARAMS,
    )
    return f(xf, offs).reshape(FT, 64, 128)


def _mlp_body(FT, g_ref, w1_ref, b1_ref, w2_ref, b2_ref, w3_ref, b3_ref,
              out_ref):
    acc = jnp.zeros((64, w1_ref.shape[1]), F32)
    for ct in range(FT):
        acc = acc + jnp.dot(g_ref[ct], w1_ref[pl.ds(ct * 128, 128), :],
                            preferred_element_type=F32)
    t = jnp.maximum(acc + b1_ref[...], 0.0)
    t2 = jnp.dot(t, w2_ref[...], preferred_element_type=F32) + b2_ref[...]
    y = jnp.dot(t2, w3_ref[...], preferred_element_type=F32) + b3_ref[...]
    out_ref[...] = y


def mlp_head(g3d, fc1_w, fc1_b, fc2_w, fc2_b, out_w_pad, out_b_pad):
    FT = g3d.shape[0]
    return pl.pallas_call(
        functools.partial(_mlp_body, FT),
        out_shape=jax.ShapeDtypeStruct((64, 128), F32),
    )(g3d, fc1_w, fc1_b.reshape(1, -1), fc2_w, fc2_b.reshape(1, -1),
      out_w_pad, out_b_pad)


def _build_A(att, oc):
    a = att[0]
    A = jnp.zeros((4 * oc, 128), F32)
    A = A.at[:2 * oc, 0].set(a[0])
    A = A.at[2 * oc:, 1].set(a[1])
    A = A.at[:2 * oc, 2].set(a[2])
    A = A.at[2 * oc:, 3].set(a[3])
    return A


def _gat_layer(h3d, att, bias, src, dst, oc):
    s4f = mm3d(h3d, _build_A(att, oc))[0, :, :4].reshape(-1)
    wrF = sc_alpha(s4f, src, dst)
    return sc_spmm(h3d, wrF, src, dst, s4f, bias, oc)


def kernel(x, edge_index, batch, W1, a1, b1, W2, a2, b2, W3, a3, b3,
           fc1_w, fc1_b, fc2_w, fc2_b, out_w, out_b):
    n, d = x.shape
    src = edge_index[0]
    dst = edge_index[1]
    x2 = _gat_layer(mm3d(x.reshape(1, n, d), W1), a1, b1, src, dst, d)
    x3 = _gat_layer(mm3d(x2, W2), a2, b2, src, dst, 2 * d)
    x4 = _gat_layer(mm3d(x3, W3), a3, b3, src, dst, 4 * d)

    offs = jnp.searchsorted(batch, jnp.arange(65, dtype=I32)).astype(I32)
    offs = jnp.concatenate([offs, jnp.full((15,), n, I32)])
    g3d = sc_segmax(x4, offs)

    out_w_pad = jnp.zeros((128, 128), F32).at[:, 0].set(out_w[:, 0])
    out_b_pad = jnp.zeros((1, 128), F32).at[0, 0].set(out_b[0])
    y = mlp_head(g3d, fc1_w, fc1_b, fc2_w, fc2_b, out_w_pad, out_b_pad)
    return y[:, :1]
